# Initial kernel scaffold; baseline (speedup 1.0000x reference)
import jax
import jax.numpy as jnp
from jax.experimental import pallas as pl


def kernel(boxes, scores, iou_threshold, max_output_boxes_per_class):
    raise NotImplementedError("write your pallas kernel here")


# speedup vs baseline: 404.6708x; 404.6708x over previous
import dataclasses
import functools

import jax
import jax.numpy as jnp
from jax import lax
from jax.experimental import pallas as pl
from jax.experimental.pallas import tpu as pltpu
from jax.experimental.pallas import tpu_sc as plsc

_B, _C, _N = 2, 8, 20000
_NROW = 16
_NCOL = 1250
_NCOLP = 1280
_SCLEN = _NROW * _NCOLP
_NG = _NCOLP // 16
_MAXOUT = 100
_OUTW = 128
_INTMAX = 2147483647


def _lexmax(v, ix, v2, i2):
    m = (v2 > v) | ((v2 == v) & (i2 < ix))
    return jnp.where(m, v2, v), jnp.where(m, i2, ix)


def _crosslane_lexmax(v, ix):
    mx = jnp.max(v)
    mi = jnp.min(jnp.where(v == mx, ix, _INTMAX))
    return mx, mi


def _nms_body(scores_hbm, boxes_hbm, thr_hbm, out_hbm,
              sc, x1r, y1r, x2r, y2r, l1v, l1i, l2v, l2i,
              ax1, ay1, ax2, ay2, aar, osel, thrv, sem):
    core = lax.axis_index("c")
    sub = lax.axis_index("s")
    active = sub < _C

    @pl.when(active)
    def _():
        b = core
        c = sub

        pltpu.async_copy(scores_hbm.at[b, c], sc, sem).wait()
        pltpu.async_copy(boxes_hbm.at[b, 0], x1r, sem).wait()
        pltpu.async_copy(boxes_hbm.at[b, 1], y1r, sem).wait()
        pltpu.async_copy(boxes_hbm.at[b, 2], x2r, sem).wait()
        pltpu.async_copy(boxes_hbm.at[b, 3], y2r, sem).wait()
        pltpu.async_copy(thr_hbm, thrv, sem).wait()

        iota = lax.iota(jnp.int32, 16)
        lane0 = iota == 0
        neg1f = jnp.full((16,), -1.0, jnp.float32)
        neg1i = jnp.full((16,), -1, jnp.int32)
        far = jnp.full((16,), 9e9, jnp.float32)
        zf = jnp.zeros((16,), jnp.float32)

        for k in range(7):
            sl = pl.ds(16 * k, 16)
            ax1[sl] = far
            ay1[sl] = far
            ax2[sl] = far
            ay2[sl] = far
            aar[sl] = zf
        for k in range(_OUTW // 16):
            osel[pl.ds(16 * k, 16)] = neg1i

        @pl.loop(0, _NG)
        def _build_l1(g):
            col0 = g * 16
            v = sc[pl.ds(col0, 16)]
            ix = col0 + iota
            for j in range(1, _NROW):
                v2 = sc[pl.ds(j * _NCOLP + col0, 16)]
                i2 = (j * _NCOL + col0) + iota
                m = v2 > v
                v = jnp.where(m, v2, v)
                ix = jnp.where(m, i2, ix)
            l1v[pl.ds(col0, 16)] = v
            l1i[pl.ds(col0, 16)] = ix

        @pl.loop(0, _NG // 16)
        def _build_l2(gg):
            g0 = gg * 16
            v = l1v[pl.ds(g0, 16)]
            ix = l1i[pl.ds(g0, 16)]
            for j in range(1, 16):
                v2 = l1v[pl.ds(j * _NG + g0, 16)]
                i2 = l1i[pl.ds(j * _NG + g0, 16)]
                v, ix = _lexmax(v, ix, v2, i2)
            l2v[pl.ds(g0, 16)] = v
            l2i[pl.ds(g0, 16)] = ix

        thr_s = jnp.max(thrv[...])
        iota_row = iota * _NCOLP
        iota_col = iota * _NCOL
        iota_grp = iota * _NG

        def cond(carry):
            nacc, done = carry
            return (nacc < _MAXOUT) & jnp.logical_not(done)

        def body(carry):
            nacc, done = carry
            v = l2v[pl.ds(0, 16)]
            ix = l2i[pl.ds(0, 16)]
            for j in range(1, _NG // 16):
                v, ix = _lexmax(v, ix, l2v[pl.ds(16 * j, 16)],
                                l2i[pl.ds(16 * j, 16)])
            mx, gi = _crosslane_lexmax(v, ix)
            alive = mx >= 0.0

            jrow = (gi * 26844) >> 25
            ccol = gi - jrow * _NCOL
            g2 = ccol - (((ccol * 6554) >> 19) * _NG)

            plsc.store_scatter(sc, [jnp.full((16,), jrow * _NCOLP + ccol)],
                               neg1f, mask=lane0 & alive)

            cv = plsc.load_gather(sc, [iota_row + ccol])
            cmx, cmi = _crosslane_lexmax(cv, iota_col + ccol)
            ccolv = jnp.full((16,), ccol)
            plsc.store_scatter(l1v, [ccolv], jnp.full((16,), cmx), mask=lane0)
            plsc.store_scatter(l1i, [ccolv], jnp.full((16,), cmi), mask=lane0)

            gidx = iota_grp + g2
            gv = plsc.load_gather(l1v, [gidx])
            gx = plsc.load_gather(l1i, [gidx])
            gmx, gmi = _crosslane_lexmax(gv, gx)
            g2v = jnp.full((16,), g2)
            plsc.store_scatter(l2v, [g2v], jnp.full((16,), gmx), mask=lane0)
            plsc.store_scatter(l2i, [g2v], jnp.full((16,), gmi), mask=lane0)

            gfull = jnp.full((16,), gi)
            bx1 = plsc.load_gather(x1r, [gfull])
            by1 = plsc.load_gather(y1r, [gfull])
            bx2 = plsc.load_gather(x2r, [gfull])
            by2 = plsc.load_gather(y2r, [gfull])
            area_a = (jnp.maximum(bx2 - bx1, 0.0)
                      * jnp.maximum(by2 - by1, 0.0))

            supmax = zf
            for k in range(7):
                sl = pl.ds(16 * k, 16)
                a1 = ax1[sl]
                b1 = ay1[sl]
                a2 = ax2[sl]
                b2 = ay2[sl]
                ab = aar[sl]
                xx1 = jnp.maximum(bx1, a1)
                yy1 = jnp.maximum(by1, b1)
                xx2 = jnp.minimum(bx2, a2)
                yy2 = jnp.minimum(by2, b2)
                inter = (jnp.maximum(xx2 - xx1, 0.0)
                         * jnp.maximum(yy2 - yy1, 0.0))
                iou = inter / jnp.maximum(area_a + ab - inter, 1e-9)
                supmax = jnp.maximum(supmax, iou)
            sup = jnp.max(supmax) > thr_s

            accept = alive & jnp.logical_not(sup)
            amask = lane0 & accept
            naccv = jnp.full((16,), nacc)
            plsc.store_scatter(osel, [naccv], gfull, mask=amask)
            plsc.store_scatter(ax1, [naccv], bx1, mask=amask)
            plsc.store_scatter(ay1, [naccv], by1, mask=amask)
            plsc.store_scatter(ax2, [naccv], bx2, mask=amask)
            plsc.store_scatter(ay2, [naccv], by2, mask=amask)
            plsc.store_scatter(aar, [naccv], area_a, mask=amask)

            return nacc + accept.astype(jnp.int32), jnp.logical_not(alive)

        lax.while_loop(cond, body, (jnp.int32(0), jnp.bool_(False)))

        pltpu.async_copy(osel, out_hbm.at[b * _C + c], sem).wait()


@jax.jit
def _nms_sc(scores_p, boxes_p, thr_arr):
    mesh = plsc.VectorSubcoreMesh(core_axis_name="c", subcore_axis_name="s",
                                  num_cores=2, num_subcores=16)
    cp = pltpu.CompilerParams()
    if "needs_layout_passes" in pltpu.CompilerParams.__dataclass_fields__:
        cp = dataclasses.replace(cp, needs_layout_passes=False)
    f = pl.kernel(
        _nms_body,
        out_type=jax.ShapeDtypeStruct((_B * _C, _OUTW), jnp.int32),
        mesh=mesh,
        scratch_types=[
            pltpu.VMEM((_SCLEN,), jnp.float32),
            pltpu.VMEM((_SCLEN,), jnp.float32),
            pltpu.VMEM((_SCLEN,), jnp.float32),
            pltpu.VMEM((_SCLEN,), jnp.float32),
            pltpu.VMEM((_SCLEN,), jnp.float32),
            pltpu.VMEM((_NCOLP,), jnp.float32),
            pltpu.VMEM((_NCOLP,), jnp.int32),
            pltpu.VMEM((_NG,), jnp.float32),
            pltpu.VMEM((_NG,), jnp.int32),
            pltpu.VMEM((112,), jnp.float32),
            pltpu.VMEM((112,), jnp.float32),
            pltpu.VMEM((112,), jnp.float32),
            pltpu.VMEM((112,), jnp.float32),
            pltpu.VMEM((112,), jnp.float32),
            pltpu.VMEM((_OUTW,), jnp.int32),
            pltpu.VMEM((16,), jnp.float32),
            pltpu.SemaphoreType.DMA,
        ],
        compiler_params=cp,
    )
    return f(scores_p, boxes_p, thr_arr)


def kernel(boxes, scores, iou_threshold, max_output_boxes_per_class):
    scores = scores.astype(jnp.float32)
    boxes = boxes.astype(jnp.float32)
    sp = scores.reshape(_B, _C, _NROW, _NCOL)
    sp = jnp.concatenate(
        [sp, jnp.full((_B, _C, _NROW, _NCOLP - _NCOL), -1.0, jnp.float32)],
        axis=-1).reshape(_B, _C, _SCLEN)
    bt = jnp.transpose(boxes, (0, 2, 1))
    bt = jnp.concatenate(
        [bt, jnp.zeros((_B, 4, _SCLEN - _N), jnp.float32)], axis=-1)
    thr_arr = jnp.full((16,), iou_threshold, jnp.float32)

    sel = _nms_sc(sp, bt, thr_arr)[:, :_MAXOUT]

    slot_ok = jnp.arange(_MAXOUT, dtype=jnp.int32) < max_output_boxes_per_class
    sel = jnp.where(slot_ok[None, :], sel, jnp.int32(-1))
    bi = jnp.repeat(jnp.arange(_B, dtype=jnp.int32), _C)
    ci = jnp.tile(jnp.arange(_C, dtype=jnp.int32), _B)
    bi = jnp.broadcast_to(bi[:, None], (_B * _C, _MAXOUT))
    ci = jnp.broadcast_to(ci[:, None], (_B * _C, _MAXOUT))
    rows = jnp.stack([bi, ci, sel], axis=-1).reshape(_B * _C * _MAXOUT, 3)
    return rows.astype(jnp.int64)
